# Initial kernel scaffold; baseline (speedup 1.0000x reference)
import jax
import jax.numpy as jnp
from jax.experimental import pallas as pl


def kernel(gt_boxes, spatial_features):
    raise NotImplementedError("write your pallas kernel here")


# speedup vs baseline: 3328.9205x; 3328.9205x over previous
import jax
import jax.numpy as jnp
from jax.experimental import pallas as pl

_NUM_CLASSES = 4
_VOXEL_X = 0.1
_VOXEL_Y = 0.1
_PCR_X = 0.0
_PCR_Y = -39.68
_NPAD = 512


def _assign_kernel(gtb_ref, off_ref, z_ref, size_ref, yaw_ref, vel_ref,
                   mask_ref):
    H = off_ref.shape[2]
    W = off_ref.shape[3]
    g = gtb_ref[0]
    cx = g[0]
    cy = g[1]
    cz = g[2]
    bw = g[3]
    bl = g[4]
    bh = g[5]
    yaw = g[6]
    vx = g[8]
    vy = g[9]
    nonzero = (jnp.abs(cx) + jnp.abs(cy) + jnp.abs(cz)) > 0.0
    gx = (cx - _PCR_X) / _VOXEL_X
    gy = (cy - _PCR_Y) / _VOXEL_Y
    gxi = jnp.floor(gx).astype(jnp.int32)
    gyi = jnp.floor(gy).astype(jnp.int32)
    xo = gx - gxi.astype(jnp.float32)
    yo = gy - gyi.astype(jnp.float32)
    inb = (gxi >= 0) & (gxi < W) & (gyi >= 0) & (gyi < H)
    valid = nonzero & inb
    lin = jnp.where(valid, gyi * W + gxi, H * W)
    ii = jax.lax.broadcasted_iota(jnp.int32, (_NPAD, _NPAD), 0)
    jj = jax.lax.broadcasted_iota(jnp.int32, (_NPAD, _NPAD), 1)
    dup = (lin[:, None] == lin[None, :]) & (jj > ii)
    conflict = jnp.any(dup, axis=1)
    survive = valid & jnp.logical_not(conflict)
    sf = survive.astype(jnp.float32)
    ycol = jax.lax.broadcasted_iota(jnp.int32, (_NPAD, H), 1)
    xcol = jax.lax.broadcasted_iota(jnp.int32, (_NPAD, W), 1)
    R = jnp.where(gyi[:, None] == ycol, sf[:, None], 0.0)
    C = (gxi[:, None] == xcol).astype(jnp.float32)
    dn = (((0,), (0,)), ((), ()))

    def scat(v):
        return jax.lax.dot_general(R * v[:, None], C, dn,
                                   preferred_element_type=jnp.float32)

    off_ref[0, 0] = scat(xo)
    off_ref[0, 1] = scat(yo)
    z_ref[0, 0] = scat(cz)
    size_ref[0, 0] = scat(bw)
    size_ref[0, 1] = scat(bl)
    size_ref[0, 2] = scat(bh)
    yaw_ref[0, 0] = scat(jnp.sin(yaw))
    yaw_ref[0, 1] = scat(jnp.cos(yaw))
    vel_ref[0, 0] = scat(vx)
    vel_ref[0, 1] = scat(vy)
    mask_ref[0, 0] = jax.lax.dot_general(R, C, dn,
                                         preferred_element_type=jnp.float32)


def kernel(gt_boxes, spatial_features):
    B, N, F = gt_boxes.shape
    H, W = spatial_features.shape[-2], spatial_features.shape[-1]
    gt = jnp.transpose(gt_boxes, (0, 2, 1))
    gt = jnp.pad(gt, ((0, 0), (0, 16 - F), (0, _NPAD - N)))

    def ospec(c):
        return pl.BlockSpec((1, c, H, W), lambda b: (b, 0, 0, 0))

    def oshape(c):
        return jax.ShapeDtypeStruct((B, c, H, W), jnp.float32)

    off, zmap, size, yawm, velm, mask = pl.pallas_call(
        _assign_kernel,
        grid=(B,),
        in_specs=[pl.BlockSpec((1, 16, _NPAD), lambda b: (b, 0, 0))],
        out_specs=[ospec(2), ospec(1), ospec(3), ospec(2), ospec(2),
                   ospec(1)],
        out_shape=[oshape(2), oshape(1), oshape(3), oshape(2), oshape(2),
                   oshape(1)],
    )(gt)
    heatmap = jnp.zeros((B, _NUM_CLASSES, H, W), jnp.float32)
    return (heatmap, off, zmap, size, yawm, velm, mask)


# speedup vs baseline: 4100.5154x; 1.2318x over previous
import jax
import jax.numpy as jnp
from jax.experimental import pallas as pl

_NUM_CLASSES = 4
_VOXEL_X = 0.1
_VOXEL_Y = 0.1
_PCR_X = 0.0
_PCR_Y = -39.68
_NPAD = 512


def _assign_kernel(gtb_ref, hm_ref, off_ref, z_ref, size_ref, yaw_ref,
                   vel_ref, mask_ref):
    H = off_ref.shape[2]
    W = off_ref.shape[3]
    g = gtb_ref[0]
    cx = g[0]
    cy = g[1]
    cz = g[2]
    bw = g[3]
    bl = g[4]
    bh = g[5]
    yaw = g[6]
    vx = g[8]
    vy = g[9]
    nonzero = (jnp.abs(cx) + jnp.abs(cy) + jnp.abs(cz)) > 0.0
    gx = (cx - _PCR_X) / _VOXEL_X
    gy = (cy - _PCR_Y) / _VOXEL_Y
    gxi = jnp.floor(gx).astype(jnp.int32)
    gyi = jnp.floor(gy).astype(jnp.int32)
    xo = gx - gxi.astype(jnp.float32)
    yo = gy - gyi.astype(jnp.float32)
    inb = (gxi >= 0) & (gxi < W) & (gyi >= 0) & (gyi < H)
    valid = nonzero & inb
    lin = jnp.where(valid, gyi * W + gxi, H * W)
    ii = jax.lax.broadcasted_iota(jnp.int32, (_NPAD, _NPAD), 0)
    jj = jax.lax.broadcasted_iota(jnp.int32, (_NPAD, _NPAD), 1)
    dup = (lin[None, :] == lin[:, None]) & (ii > jj)
    conflict = jnp.any(dup, axis=0)
    survive = valid & jnp.logical_not(conflict)
    sf = survive.astype(jnp.float32)
    ycol = jax.lax.broadcasted_iota(jnp.int32, (_NPAD, H), 1)
    xcol = jax.lax.broadcasted_iota(jnp.int32, (_NPAD, W), 1)
    R = jnp.where(gyi[:, None] == ycol, sf[:, None], 0.0)
    C = (gxi[:, None] == xcol).astype(jnp.float32)
    dn = (((0,), (0,)), ((), ()))

    def scat(v):
        return jax.lax.dot_general(R * v[:, None], C, dn,
                                   preferred_element_type=jnp.float32)

    hm_ref[...] = jnp.zeros_like(hm_ref)
    off_ref[0, 0] = scat(xo)
    off_ref[0, 1] = scat(yo)
    z_ref[0, 0] = scat(cz)
    size_ref[0, 0] = scat(bw)
    size_ref[0, 1] = scat(bl)
    size_ref[0, 2] = scat(bh)
    yaw_ref[0, 0] = scat(jnp.sin(yaw))
    yaw_ref[0, 1] = scat(jnp.cos(yaw))
    vel_ref[0, 0] = scat(vx)
    vel_ref[0, 1] = scat(vy)
    mask_ref[0, 0] = jax.lax.dot_general(R, C, dn,
                                         preferred_element_type=jnp.float32)


def kernel(gt_boxes, spatial_features):
    B, N, F = gt_boxes.shape
    H, W = spatial_features.shape[-2], spatial_features.shape[-1]
    gt = jnp.transpose(gt_boxes, (0, 2, 1))
    gt = jnp.pad(gt, ((0, 0), (0, 16 - F), (0, _NPAD - N)))

    def ospec(c):
        return pl.BlockSpec((1, c, H, W), lambda b: (b, 0, 0, 0))

    def oshape(c):
        return jax.ShapeDtypeStruct((B, c, H, W), jnp.float32)

    heatmap, off, zmap, size, yawm, velm, mask = pl.pallas_call(
        _assign_kernel,
        grid=(B,),
        in_specs=[pl.BlockSpec((1, 16, _NPAD), lambda b: (b, 0, 0))],
        out_specs=[ospec(_NUM_CLASSES), ospec(2), ospec(1), ospec(3),
                   ospec(2), ospec(2), ospec(1)],
        out_shape=[oshape(_NUM_CLASSES), oshape(2), oshape(1), oshape(3),
                   oshape(2), oshape(2), oshape(1)],
    )(gt)
    return (heatmap, off, zmap, size, yawm, velm, mask)
